# baseline (speedup 1.0000x reference)
import functools

import numpy as np
import jax
import jax.numpy as jnp
from jax import lax
from jax.experimental import pallas as pl
from jax.experimental.pallas import tpu as pltpu
from jax.experimental.pallas import tpu_sc as plsc

_MAX_SEQ = 200
_D = 64
_BATCH = 4096
_SEQ = 200

_info = plsc.get_sparse_core_info()
_NC = _info.num_cores
_NS = _info.num_subcores
_NW = _NC * _NS

_BT = _BATCH // _NW
_LANES = 16
_NBUF = 3
_BU = 8
_TP = _BT + 1


def _pe_table() -> np.ndarray:
    row_vec = np.zeros(_D, dtype=np.float64)
    row_vec[::2] = np.arange(0, _D, 2) / _D
    row_vec[1::2] = np.arange(0, _D, 2) / _D
    row_vec = 10000.0 ** row_vec
    col_vec = np.arange(0, _MAX_SEQ, 1, dtype=np.float64).reshape(-1, 1)
    pe = col_vec / row_vec
    pe[:, ::2] = np.sin(pe[:, ::2])
    pe[:, 1::2] = np.cos(pe[:, 1::2])
    return pe.astype(np.float32).reshape(-1)


@functools.partial(
    pl.kernel,
    mesh=plsc.VectorSubcoreMesh(core_axis_name="c", subcore_axis_name="s"),
    out_type=jax.ShapeDtypeStruct((_SEQ, _D // 8, _NW, 8, _BT), jnp.float32),
    compiler_params=pltpu.CompilerParams(
        use_tc_tiling_on_sc=False, needs_layout_passes=False),
    scratch_types=[
        pltpu.VMEM((_BT * _SEQ,), jnp.int32),
        pltpu.VMEM((_SEQ * _D,), jnp.float32),
        pltpu.VMEM((_BT,), jnp.int32),
        pltpu.VMEM((_BT,), jnp.int32),
        pltpu.VMEM((_BT,), jnp.int32),
        pltpu.VMEM((_BT, _D), jnp.float32),
        pltpu.VMEM((_BT, _D), jnp.float32),
        pltpu.VMEM((_BT, _D), jnp.float32),
        pltpu.VMEM((_D, _TP), jnp.float32),
        pltpu.VMEM((_D, _TP), jnp.float32),
        pltpu.VMEM((_D, _TP), jnp.float32),
        pltpu.SemaphoreType.DMA,
        pltpu.SemaphoreType.DMA,
        pltpu.SemaphoreType.DMA,
        pltpu.SemaphoreType.DMA,
        pltpu.SemaphoreType.DMA,
        pltpu.SemaphoreType.DMA,
    ],
)
def _sc_embed_t(x_hbm, table_hbm, pe_hbm, out_hbm, idx_v, pe_v,
                c0, c1, c2, r0, r1, r2, t0, t1, t2,
                g0, g1, g2, s0, s1, s2):
    cols = (c0, c1, c2)
    rows = (r0, r1, r2)
    tbufs = (t0, t1, t2)
    gsems = (g0, g1, g2)
    ssems = (s0, s1, s2)
    wid = lax.axis_index("s") * _NC + lax.axis_index("c")
    pltpu.sync_copy(x_hbm.at[pl.ds(wid * (_BT * _SEQ), _BT * _SEQ)], idx_v)
    pltpu.sync_copy(pe_hbm, pe_v)

    iota = lax.iota(jnp.int32, _LANES)
    iota_s = iota * _SEQ
    d_rows = [iota + c * _LANES for c in range(_D // _LANES)]

    def build_idxcol(s, k):
        for bg in range(_BT // _LANES):
            v = plsc.load_gather(
                idx_v, [iota_s + (bg * (_LANES * _SEQ) + s)])
            cols[k][pl.ds(bg * _LANES, _LANES)] = v

    def issue_gather(k):
        pltpu.async_copy(table_hbm.at[cols[k]], rows[k], gsems[k])

    def drain_gather(k):
        pltpu.make_async_copy(table_hbm.at[cols[k]], rows[k], gsems[k]).wait()

    def transpose_add(s, k):
        pe16 = [pe_v[pl.ds(s * _D + c * _LANES, _LANES)]
                for c in range(_D // _LANES)]

        @plsc.parallel_loop(0, _BT, step=1, unroll=_BU)
        def body(b):
            b_vec = jnp.full((_LANES,), b, dtype=jnp.int32)
            for c in range(_D // _LANES):
                v = rows[k][b, pl.ds(c * _LANES, _LANES)] + pe16[c]
                plsc.store_scatter(tbufs[k], [d_rows[c], b_vec], v)

    def issue_store(s, k):
        for dt in range(_D // 8):
            pltpu.async_copy(tbufs[k].at[pl.ds(dt * 8, 8), pl.ds(0, _BT)],
                             out_hbm.at[s, dt, wid], ssems[k])

    def wait_store(s_prev, k):
        for dt in range(_D // 8):
            pltpu.make_async_copy(
                tbufs[k].at[pl.ds(dt * 8, 8), pl.ds(0, _BT)],
                out_hbm.at[s_prev, dt, wid], ssems[k]).wait()

    def process(s, k, tstore_wait, issue_next):
        drain_gather(k)
        if issue_next:
            k2 = (k + 2) % _NBUF
            build_idxcol(s + 2, k2)
            issue_gather(k2)
        if tstore_wait:
            wait_store(s - _NBUF, k)
        transpose_add(s, k)
        issue_store(s, k)

    build_idxcol(0, 0)
    issue_gather(0)
    build_idxcol(1, 1)
    issue_gather(1)
    for s in range(_NBUF):
        process(s, s % _NBUF, tstore_wait=False, issue_next=True)

    n_trips = (_SEQ - _NBUF - 2) // _NBUF

    def trip(i, carry):
        sbase = _NBUF + i * _NBUF
        for k0 in range(_NBUF):
            process(sbase + k0, k0, tstore_wait=True, issue_next=True)
        return carry

    lax.fori_loop(0, n_trips, trip, 0)
    for s in range(_NBUF + n_trips * _NBUF, _SEQ):
        process(s, s % _NBUF, tstore_wait=True, issue_next=False)
    for s in range(_SEQ - _NBUF, _SEQ):
        wait_store(s, s % _NBUF)


def kernel(x, table):
    pe = jnp.asarray(_pe_table())
    out_t = _sc_embed_t(x.reshape(-1), table, pe)
    return out_t.transpose(2, 4, 0, 1, 3).reshape(_BATCH, _SEQ, _D)


# speedup vs baseline: 1.1067x; 1.1067x over previous
import functools

import numpy as np
import jax
import jax.numpy as jnp
from jax import lax
from jax.experimental import pallas as pl
from jax.experimental.pallas import tpu as pltpu
from jax.experimental.pallas import tpu_sc as plsc

_MAX_SEQ = 200
_D = 64
_BATCH = 4096
_SEQ = 200

_info = plsc.get_sparse_core_info()
_NC = _info.num_cores
_NS = _info.num_subcores
_NW = _NC * _NS

_BT = _BATCH // _NW
_LANES = 16
_NBUF = 3
_BU = 4
_TP = _BT + 1


def _pe_table() -> np.ndarray:
    row_vec = np.zeros(_D, dtype=np.float64)
    row_vec[::2] = np.arange(0, _D, 2) / _D
    row_vec[1::2] = np.arange(0, _D, 2) / _D
    row_vec = 10000.0 ** row_vec
    col_vec = np.arange(0, _MAX_SEQ, 1, dtype=np.float64).reshape(-1, 1)
    pe = col_vec / row_vec
    pe[:, ::2] = np.sin(pe[:, ::2])
    pe[:, 1::2] = np.cos(pe[:, 1::2])
    return pe.astype(np.float32).reshape(-1)


@functools.partial(
    pl.kernel,
    mesh=plsc.VectorSubcoreMesh(core_axis_name="c", subcore_axis_name="s"),
    out_type=jax.ShapeDtypeStruct((_SEQ, _D // 8, _NW, 8, _BT), jnp.float32),
    compiler_params=pltpu.CompilerParams(
        use_tc_tiling_on_sc=False, needs_layout_passes=False),
    scratch_types=[
        pltpu.VMEM((_BT * _SEQ,), jnp.int32),
        pltpu.VMEM((_SEQ * _D,), jnp.float32),
        pltpu.VMEM((_BT,), jnp.int32),
        pltpu.VMEM((_BT,), jnp.int32),
        pltpu.VMEM((_BT,), jnp.int32),
        pltpu.VMEM((_BT, _D), jnp.float32),
        pltpu.VMEM((_BT, _D), jnp.float32),
        pltpu.VMEM((_BT, _D), jnp.float32),
        pltpu.VMEM((_D // 8, 8, _TP), jnp.float32),
        pltpu.VMEM((_D // 8, 8, _TP), jnp.float32),
        pltpu.VMEM((_D // 8, 8, _TP), jnp.float32),
        pltpu.SemaphoreType.DMA,
        pltpu.SemaphoreType.DMA,
        pltpu.SemaphoreType.DMA,
        pltpu.SemaphoreType.DMA,
        pltpu.SemaphoreType.DMA,
        pltpu.SemaphoreType.DMA,
    ],
)
def _sc_embed_t(x_hbm, table_hbm, pe_hbm, out_hbm, idx_v, pe_v,
                c0, c1, c2, r0, r1, r2, t0, t1, t2,
                g0, g1, g2, s0, s1, s2):
    cols = (c0, c1, c2)
    rows = (r0, r1, r2)
    tbufs = (t0, t1, t2)
    gsems = (g0, g1, g2)
    ssems = (s0, s1, s2)
    wid = lax.axis_index("s") * _NC + lax.axis_index("c")
    pltpu.sync_copy(x_hbm.at[pl.ds(wid * (_BT * _SEQ), _BT * _SEQ)], idx_v)
    pltpu.sync_copy(pe_hbm, pe_v)

    iota = lax.iota(jnp.int32, _LANES)
    iota_s = iota * _SEQ
    d_tiles = [(iota + c * _LANES) // 8 for c in range(_D // _LANES)]
    d_subs = [(iota + c * _LANES) % 8 for c in range(_D // _LANES)]

    def build_idxcol(s, k):
        for bg in range(_BT // _LANES):
            v = plsc.load_gather(
                idx_v, [iota_s + (bg * (_LANES * _SEQ) + s)])
            cols[k][pl.ds(bg * _LANES, _LANES)] = v

    def issue_gather(k):
        pltpu.async_copy(table_hbm.at[cols[k]], rows[k], gsems[k])

    def drain_gather(k):
        pltpu.make_async_copy(table_hbm.at[cols[k]], rows[k], gsems[k]).wait()

    def transpose_add(s, k):
        pe16 = [pe_v[pl.ds(s * _D + c * _LANES, _LANES)]
                for c in range(_D // _LANES)]

        @plsc.parallel_loop(0, _BT, step=1, unroll=_BU)
        def body(b):
            b_vec = jnp.full((_LANES,), b, dtype=jnp.int32)
            for c in range(_D // _LANES):
                v = rows[k][b, pl.ds(c * _LANES, _LANES)] + pe16[c]
                plsc.store_scatter(
                    tbufs[k], [d_tiles[c], d_subs[c], b_vec], v)

    def issue_store(s, k):
        pltpu.async_copy(tbufs[k].at[:, :, pl.ds(0, _BT)],
                         out_hbm.at[s, :, wid], ssems[k])

    def wait_store(s_prev, k):
        pltpu.make_async_copy(
            tbufs[k].at[:, :, pl.ds(0, _BT)],
            out_hbm.at[s_prev, :, wid], ssems[k]).wait()

    def process(s, k, tstore_wait, issue_next):
        drain_gather(k)
        if issue_next:
            k2 = (k + 2) % _NBUF
            build_idxcol(s + 2, k2)
            issue_gather(k2)
        if tstore_wait:
            wait_store(s - _NBUF, k)
        transpose_add(s, k)
        issue_store(s, k)

    build_idxcol(0, 0)
    issue_gather(0)
    build_idxcol(1, 1)
    issue_gather(1)
    for s in range(_NBUF):
        process(s, s % _NBUF, tstore_wait=False, issue_next=True)

    n_trips = (_SEQ - _NBUF - 2) // _NBUF

    def trip(i, carry):
        sbase = _NBUF + i * _NBUF
        for k0 in range(_NBUF):
            process(sbase + k0, k0, tstore_wait=True, issue_next=True)
        return carry

    lax.fori_loop(0, n_trips, trip, 0)
    for s in range(_NBUF + n_trips * _NBUF, _SEQ):
        process(s, s % _NBUF, tstore_wait=True, issue_next=False)
    for s in range(_SEQ - _NBUF, _SEQ):
        wait_store(s, s % _NBUF)


def kernel(x, table):
    pe = jnp.asarray(_pe_table())
    out_t = _sc_embed_t(x.reshape(-1), table, pe)
    return out_t.transpose(2, 4, 0, 1, 3).reshape(_BATCH, _SEQ, _D)


# speedup vs baseline: 1.1410x; 1.0310x over previous
import functools

import numpy as np
import jax
import jax.numpy as jnp
from jax import lax
from jax.experimental import pallas as pl
from jax.experimental.pallas import tpu as pltpu
from jax.experimental.pallas import tpu_sc as plsc

_MAX_SEQ = 200
_D = 64
_BATCH = 4096
_SEQ = 200

_info = plsc.get_sparse_core_info()
_NC = _info.num_cores
_NS = _info.num_subcores
_NW = _NC * _NS

_BT = _BATCH // _NW
_LANES = 16
_NBUF = 3
_BU = 4
_TP = _BT + 1


def _pe_table() -> np.ndarray:
    row_vec = np.zeros(_D, dtype=np.float64)
    row_vec[::2] = np.arange(0, _D, 2) / _D
    row_vec[1::2] = np.arange(0, _D, 2) / _D
    row_vec = 10000.0 ** row_vec
    col_vec = np.arange(0, _MAX_SEQ, 1, dtype=np.float64).reshape(-1, 1)
    pe = col_vec / row_vec
    pe[:, ::2] = np.sin(pe[:, ::2])
    pe[:, 1::2] = np.cos(pe[:, 1::2])
    return pe.astype(np.float32).reshape(-1)


@functools.partial(
    pl.kernel,
    mesh=plsc.VectorSubcoreMesh(core_axis_name="c", subcore_axis_name="s"),
    out_type=jax.ShapeDtypeStruct((_SEQ, _D // 8, _NW, 8, _BT), jnp.float32),
    compiler_params=pltpu.CompilerParams(
        use_tc_tiling_on_sc=False, needs_layout_passes=False),
    scratch_types=[
        pltpu.VMEM((_BT * _SEQ,), jnp.int32),
        pltpu.VMEM((_SEQ * _D,), jnp.float32),
        pltpu.VMEM((_BT,), jnp.int32),
        pltpu.VMEM((_BT,), jnp.int32),
        pltpu.VMEM((_BT,), jnp.int32),
        pltpu.VMEM((_BT, _D), jnp.float32),
        pltpu.VMEM((_BT, _D), jnp.float32),
        pltpu.VMEM((_BT, _D), jnp.float32),
        pltpu.VMEM((_D // 8, 8, _TP), jnp.float32),
        pltpu.VMEM((_D // 8, 8, _TP), jnp.float32),
        pltpu.VMEM((_D // 8, 8, _TP), jnp.float32),
        pltpu.SemaphoreType.DMA,
        pltpu.SemaphoreType.DMA,
        pltpu.SemaphoreType.DMA,
        pltpu.SemaphoreType.DMA,
        pltpu.SemaphoreType.DMA,
        pltpu.SemaphoreType.DMA,
    ],
)
def _sc_embed_t(x_hbm, table_hbm, pe_hbm, out_hbm, idx_v, pe_v,
                c0, c1, c2, r0, r1, r2, t0, t1, t2,
                g0, g1, g2, s0, s1, s2):
    cols = (c0, c1, c2)
    rows = (r0, r1, r2)
    tbufs = (t0, t1, t2)
    gsems = (g0, g1, g2)
    ssems = (s0, s1, s2)
    wid = lax.axis_index("s") * _NC + lax.axis_index("c")
    pltpu.sync_copy(x_hbm.at[pl.ds(wid * (_BT * _SEQ), _BT * _SEQ)], idx_v)
    pltpu.sync_copy(pe_hbm, pe_v)

    iota = lax.iota(jnp.int32, _LANES)
    iota_s = iota * _SEQ
    d_tiles = [(iota + c * _LANES) // 8 for c in range(_D // _LANES)]
    d_subs = [(iota + c * _LANES) % 8 for c in range(_D // _LANES)]

    def build_idxcol(s, k):
        for bg in range(_BT // _LANES):
            v = plsc.load_gather(
                idx_v, [iota_s + (bg * (_LANES * _SEQ) + s)])
            cols[k][pl.ds(bg * _LANES, _LANES)] = v

    def issue_gather(k):
        pltpu.async_copy(table_hbm.at[cols[k]], rows[k], gsems[k])

    def drain_gather(k):
        pltpu.make_async_copy(table_hbm.at[cols[k]], rows[k], gsems[k]).wait()

    def transpose_add(s, k):
        pe16 = [pe_v[pl.ds(s * _D + c * _LANES, _LANES)]
                for c in range(_D // _LANES)]

        @plsc.parallel_loop(0, _BT, step=1, unroll=_BU)
        def body(b):
            b_vec = jnp.full((_LANES,), b, dtype=jnp.int32)
            for c in range(_D // _LANES):
                v = rows[k][b, pl.ds(c * _LANES, _LANES)] + pe16[c]
                plsc.store_scatter(
                    tbufs[k], [d_tiles[c], d_subs[c], b_vec], v)

    def issue_store(s, k):
        pltpu.async_copy(tbufs[k].at[:, :, pl.ds(0, _BT)],
                         out_hbm.at[s, :, wid], ssems[k])

    def wait_store(s_prev, k):
        pltpu.make_async_copy(
            tbufs[k].at[:, :, pl.ds(0, _BT)],
            out_hbm.at[s_prev, :, wid], ssems[k]).wait()

    def process(s, k, tstore_wait, issue_next):
        if issue_next:
            k2 = (k + 2) % _NBUF
            build_idxcol(s + 2, k2)
            issue_gather(k2)
        drain_gather(k)
        if tstore_wait:
            wait_store(s - _NBUF, k)
        transpose_add(s, k)
        issue_store(s, k)

    build_idxcol(0, 0)
    issue_gather(0)
    build_idxcol(1, 1)
    issue_gather(1)
    for s in range(_NBUF):
        process(s, s % _NBUF, tstore_wait=False, issue_next=True)

    n_trips = (_SEQ - _NBUF - 2) // _NBUF

    def trip(i, carry):
        sbase = _NBUF + i * _NBUF
        for k0 in range(_NBUF):
            process(sbase + k0, k0, tstore_wait=True, issue_next=True)
        return carry

    lax.fori_loop(0, n_trips, trip, 0)
    for s in range(_NBUF + n_trips * _NBUF, _SEQ):
        process(s, s % _NBUF, tstore_wait=True, issue_next=False)
    for s in range(_SEQ - _NBUF, _SEQ):
        wait_store(s, s % _NBUF)


def kernel(x, table):
    pe = jnp.asarray(_pe_table())
    out_t = _sc_embed_t(x.reshape(-1), table, pe)
    return out_t.transpose(2, 4, 0, 1, 3).reshape(_BATCH, _SEQ, _D)
